# Initial kernel scaffold; baseline (speedup 1.0000x reference)
import jax
import jax.numpy as jnp
from jax.experimental import pallas as pl


def kernel(x_seq, edge_index_seq, target_indices, W_proj, b_proj, W_g1, b_g1, W_g2, b_g2, W_ih, W_hh, b_ih, b_hh, W_att, b_att, W_p1, b_p1, W_p2, b_p2):
    raise NotImplementedError("write your pallas kernel here")


# speedup vs baseline: 12.7335x; 12.7335x over previous
import functools

import jax
import jax.numpy as jnp
from jax import lax
from jax.experimental import pallas as pl
from jax.experimental.pallas import tpu as pltpu
from jax.experimental.pallas import tpu_sc as plsc

T, N, E, D = 4, 10000, 320000, 128
P, G, R, B = 128, 128, 128, 1024

NP = 10240
EP = 327680
NPAD = NP - N
NW = 32
EPW = EP // NW
CH = 128
NCH = EPW // CH
RT = NP // 16

RB = 256
NRT = NP // RB

@functools.cache
def _mesh():
    return plsc.VectorSubcoreMesh(core_axis_name="c", subcore_axis_name="s",
                                  num_cores=2, num_subcores=16)
_HIGH = lax.Precision.HIGHEST


def _dot(a, b):
    return lax.dot_general(a, b, (((1,), (0,)), ((), ())),
                           precision=_HIGH, preferred_element_type=jnp.float32)


def _deg_body(dst_hbm, out_hbm, acc, idx_d, ones_v, zero_v, sem):
    del sem
    c = lax.axis_index("c")
    s = lax.axis_index("s")
    wid = s * 2 + c
    row0 = s * RT

    @pl.loop(0, CH // 16)
    def _init_ones(i):
        ones_v[pl.ds(i * 16, 16)] = jnp.ones((16,), jnp.float32)

    @pl.loop(0, RT // 16)
    def _init_zero(i):
        zero_v[pl.ds(i * 16, 16)] = jnp.zeros((16,), jnp.float32)

    for t in range(T):
        pltpu.sync_copy(zero_v, acc.at[pl.ds(row0, RT)])
        plsc.subcore_barrier()

        @pl.loop(0, NCH)
        def _scatter(j):
            off = t * EP + wid * EPW + j * CH
            pltpu.sync_copy(dst_hbm.at[pl.ds(off, CH)], idx_d)
            pltpu.sync_copy(ones_v, acc.at[idx_d], add=True)

        plsc.subcore_barrier()
        pltpu.sync_copy(acc.at[pl.ds(row0, RT)],
                        out_hbm.at[t, c, pl.ds(row0, RT)])


@functools.cache
def _deg_call():
    return pl.kernel(
        _deg_body,
        out_type=jax.ShapeDtypeStruct((T, 2, NP), jnp.float32),
        mesh=_mesh(),
        scratch_types=[
            pltpu.VMEM_SHARED((NP,), jnp.float32),
            pltpu.VMEM((CH,), jnp.int32),
            pltpu.VMEM((CH,), jnp.float32),
            pltpu.VMEM((RT,), jnp.float32),
            pltpu.SemaphoreType.DMA,
        ],
    )


def _conv_body(y_hbm, src_hbm, dst_hbm, out_hbm, acc, idx_s, idx_d, rows,
               zrows, sem):
    c = lax.axis_index("c")
    s = lax.axis_index("s")
    wid = s * 2 + c
    row0 = s * RT

    @pl.loop(0, CH * D // 16)
    def _init_zero(i):
        zrows[i // (D // 16), pl.ds((i % (D // 16)) * 16, 16)] = (
            jnp.zeros((16,), jnp.float32))

    @pl.loop(0, RT // CH)
    def _zero_acc(k):
        pltpu.sync_copy(zrows, acc.at[pl.ds(row0 + k * CH, CH)])

    plsc.subcore_barrier()

    base = wid * EPW

    @pl.loop(0, NCH)
    def _edges(j):
        off = base + j * CH
        pltpu.sync_copy(src_hbm.at[pl.ds(off, CH)], idx_s)
        pltpu.sync_copy(dst_hbm.at[pl.ds(off, CH)], idx_d)
        pltpu.async_copy(y_hbm.at[idx_s], rows, sem).wait()
        pltpu.sync_copy(rows, acc.at[idx_d], add=True)

    plsc.subcore_barrier()

    @pl.loop(0, RT // CH)
    def _copy_out(k):
        r = row0 + k * CH
        pltpu.sync_copy(acc.at[pl.ds(r, CH)], out_hbm.at[c, pl.ds(r, CH)])


@functools.cache
def _conv_call():
    return pl.kernel(
        _conv_body,
        out_type=jax.ShapeDtypeStruct((2, NP, D), jnp.float32),
        mesh=_mesh(),
        scratch_types=[
            pltpu.VMEM_SHARED((NP, D), jnp.float32),
            pltpu.VMEM((CH,), jnp.int32),
            pltpu.VMEM((CH,), jnp.int32),
            pltpu.VMEM((CH, D), jnp.float32),
            pltpu.VMEM((CH, D), jnp.float32),
            pltpu.SemaphoreType.DMA,
        ],
    )

BW = B // NW


def _tgt_gather_body(h10, h11, h12, h13, h20, h21, h22, h23, tgt_hbm,
                     out_hbm, idx_v, rows, sem):
    c = lax.axis_index("c")
    s = lax.axis_index("s")
    wid = s * 2 + c
    b0 = wid * BW
    pltpu.sync_copy(tgt_hbm.at[pl.ds(b0, BW)], idx_v)
    planes = [h10, h20, h11, h21, h12, h22, h13, h23]
    for p, h in enumerate(planes):
        pltpu.async_copy(h.at[idx_v], rows, sem).wait()
        pltpu.sync_copy(rows, out_hbm.at[p, pl.ds(b0, BW)])


@functools.cache
def _tgt_gather_call():
    return pl.kernel(
        _tgt_gather_body,
        out_type=jax.ShapeDtypeStruct((2 * T, B, G), jnp.float32),
        mesh=_mesh(),
        scratch_types=[
            pltpu.VMEM((BW,), jnp.int32),
            pltpu.VMEM((BW, G), jnp.float32),
            pltpu.SemaphoreType.DMA,
        ],
    )


def _proj_body(x_ref, degp_ref, wp_ref, bp_ref, wg1_ref, y1_ref):
    x = x_ref[0]
    h = jnp.maximum(_dot(x, wp_ref[...]) + bp_ref[...], 0.0)
    xw = _dot(h, wg1_ref[...])
    deg = 1.0 + degp_ref[0, 0, :] + degp_ref[0, 1, :]
    dinv = lax.rsqrt(deg)[:, None]
    y1_ref[...] = xw * dinv


def _proj_call(t, xp, degp, wp, bp, wg1):
    return pl.pallas_call(
        _proj_body,
        grid=(NRT,),
        in_specs=[
            pl.BlockSpec((1, RB, D), lambda r, _t=t: (_t, r, 0)),
            pl.BlockSpec((1, 2, RB), lambda r, _t=t: (_t, 0, r)),
            pl.BlockSpec((D, P), lambda r: (0, 0)),
            pl.BlockSpec((1, P), lambda r: (0, 0)),
            pl.BlockSpec((P, G), lambda r: (0, 0)),
        ],
        out_specs=pl.BlockSpec((RB, G), lambda r: (r, 0)),
        out_shape=jax.ShapeDtypeStruct((NP, G), jnp.float32),
    )(xp, degp, wp, bp, wg1)


def _mid_body(p_ref, y1_ref, degp_ref, bg1_ref, wg2_ref, h1_ref, y2_ref):
    ssum = p_ref[0] + p_ref[1] + y1_ref[...]
    deg = 1.0 + degp_ref[0, 0, :] + degp_ref[0, 1, :]
    dinv = lax.rsqrt(deg)[:, None]
    h1 = jnp.maximum(ssum * dinv + bg1_ref[...], 0.0)
    h1_ref[...] = h1
    y2_ref[...] = _dot(h1, wg2_ref[...]) * dinv


def _mid_call(t, p, y1, degp, bg1, wg2):
    return pl.pallas_call(
        _mid_body,
        grid=(NRT,),
        in_specs=[
            pl.BlockSpec((2, RB, G), lambda r: (0, r, 0)),
            pl.BlockSpec((RB, G), lambda r: (r, 0)),
            pl.BlockSpec((1, 2, RB), lambda r, _t=t: (_t, 0, r)),
            pl.BlockSpec((1, G), lambda r: (0, 0)),
            pl.BlockSpec((G, G), lambda r: (0, 0)),
        ],
        out_specs=[
            pl.BlockSpec((RB, G), lambda r: (r, 0)),
            pl.BlockSpec((RB, G), lambda r: (r, 0)),
        ],
        out_shape=[
            jax.ShapeDtypeStruct((NP, G), jnp.float32),
            jax.ShapeDtypeStruct((NP, G), jnp.float32),
        ],
    )(p, y1, degp, bg1, wg2)


def _post_body(q_ref, y2_ref, degp_ref, bg2_ref, h2_ref):
    ssum = q_ref[0] + q_ref[1] + y2_ref[...]
    deg = 1.0 + degp_ref[0, 0, :] + degp_ref[0, 1, :]
    dinv = lax.rsqrt(deg)[:, None]
    h2_ref[...] = jnp.maximum(ssum * dinv + bg2_ref[...], 0.0)


def _post_call(t, q, y2, degp, bg2):
    return pl.pallas_call(
        _post_body,
        grid=(NRT,),
        in_specs=[
            pl.BlockSpec((2, RB, G), lambda r: (0, r, 0)),
            pl.BlockSpec((RB, G), lambda r: (r, 0)),
            pl.BlockSpec((1, 2, RB), lambda r, _t=t: (_t, 0, r)),
            pl.BlockSpec((1, G), lambda r: (0, 0)),
        ],
        out_specs=pl.BlockSpec((RB, G), lambda r: (r, 0)),
        out_shape=jax.ShapeDtypeStruct((NP, G), jnp.float32),
    )(q, y2, degp, bg2)


RBG = 256


def _gru_body(g_ref, wia_ref, wib_ref, whh_ref, bih_ref, bhh_ref, watt_ref,
              batt_ref, wp1_ref, bp1_ref, wp2_ref, bp2_ref, o_ref):
    h = jnp.zeros((RBG, R), jnp.float32)
    wia = wia_ref[...]
    wib = wib_ref[...]
    whh = whh_ref[...]
    hs, es = [], []
    for t in range(T):
        gi = _dot(g_ref[2 * t], wia) + _dot(g_ref[2 * t + 1], wib) + bih_ref[...]
        gh = _dot(h, whh) + bhh_ref[...]
        rg = jax.nn.sigmoid(gi[:, :R] + gh[:, :R])
        zg = jax.nn.sigmoid(gi[:, R:2 * R] + gh[:, R:2 * R])
        ng = jnp.tanh(gi[:, 2 * R:] + rg * gh[:, 2 * R:])
        h = (1.0 - zg) * ng + zg * h
        hs.append(h)
        es.append(jnp.tanh(
            jnp.sum(h * watt_ref[...], axis=1, keepdims=True) + batt_ref[0, 0]))
    m = jnp.maximum(jnp.maximum(es[0], es[1]), jnp.maximum(es[2], es[3]))
    ws = [jnp.exp(s - m) for s in es]
    den = ws[0] + ws[1] + ws[2] + ws[3]
    rep = (hs[0] * ws[0] + hs[1] * ws[1] + hs[2] * ws[2] + hs[3] * ws[3]) / den
    hp = jnp.maximum(_dot(rep, wp1_ref[...]) + bp1_ref[...], 0.0)
    o_ref[...] = jnp.sum(hp * wp2_ref[...], axis=1, keepdims=True) + bp2_ref[0, 0]


def _gru_call(g, wia, wib, whh, bih, bhh, watt, batt, wp1, bp1, wp2, bp2):
    return pl.pallas_call(
        _gru_body,
        grid=(B // RBG,),
        in_specs=[
            pl.BlockSpec((2 * T, RBG, G), lambda r: (0, r, 0)),
            pl.BlockSpec((G, 3 * R), lambda r: (0, 0)),
            pl.BlockSpec((G, 3 * R), lambda r: (0, 0)),
            pl.BlockSpec((R, 3 * R), lambda r: (0, 0)),
            pl.BlockSpec((1, 3 * R), lambda r: (0, 0)),
            pl.BlockSpec((1, 3 * R), lambda r: (0, 0)),
            pl.BlockSpec((1, R), lambda r: (0, 0)),
            pl.BlockSpec((1, 1), lambda r: (0, 0)),
            pl.BlockSpec((R, 16), lambda r: (0, 0)),
            pl.BlockSpec((1, 16), lambda r: (0, 0)),
            pl.BlockSpec((1, 16), lambda r: (0, 0)),
            pl.BlockSpec((1, 1), lambda r: (0, 0)),
        ],
        out_specs=pl.BlockSpec((RBG, 1), lambda r: (r, 0)),
        out_shape=jax.ShapeDtypeStruct((B, 1), jnp.float32),
    )(g, wia, wib, whh, bih, bhh, watt, batt, wp1, bp1, wp2, bp2)


def kernel(x_seq, edge_index_seq, target_indices, W_proj, b_proj, W_g1, b_g1,
           W_g2, b_g2, W_ih, W_hh, b_ih, b_hh, W_att, b_att, W_p1, b_p1,
           W_p2, b_p2):
    xp = jnp.pad(x_seq, ((0, 0), (0, NP - N), (0, 0)))

    src = edge_index_seq[:, 0, :]
    dst = edge_index_seq[:, 1, :]
    padi = (N + (jnp.arange(EP - E, dtype=jnp.int32) % NPAD))[None, :]
    pads = jnp.broadcast_to(padi, (T, EP - E))
    srcp = jnp.concatenate([src, pads], axis=1)
    dstp = jnp.concatenate([dst, pads], axis=1)

    degp = _deg_call()(dstp.reshape(-1))

    bp = b_proj.reshape(1, P)
    bg1 = b_g1.reshape(1, G)
    bg2 = b_g2.reshape(1, G)

    h1s, h2s = [], []
    for t in range(T):
        y1 = _proj_call(t, xp, degp, W_proj, bp, W_g1)
        p = _conv_call()(y1, srcp[t], dstp[t])
        h1, y2 = _mid_call(t, p, y1, degp, bg1, W_g2)
        q = _conv_call()(y2, srcp[t], dstp[t])
        h2 = _post_call(t, q, y2, degp, bg2)
        h1s.append(h1)
        h2s.append(h2)

    g = _tgt_gather_call()(h1s[0], h1s[1], h1s[2], h1s[3],
                           h2s[0], h2s[1], h2s[2], h2s[3], target_indices)

    wihT = W_ih.T
    out = _gru_call(
        g, wihT[:G], wihT[G:], W_hh.T,
        b_ih.reshape(1, 3 * R), b_hh.reshape(1, 3 * R),
        W_att.T, b_att.reshape(1, 1),
        W_p1, b_p1.reshape(1, 16), W_p2.T, b_p2.reshape(1, 1))
    return out


# speedup vs baseline: 24.1054x; 1.8931x over previous
import functools

import jax
import jax.numpy as jnp
from jax import lax
from jax.experimental import pallas as pl
from jax.experimental.pallas import tpu as pltpu
from jax.experimental.pallas import tpu_sc as plsc

T, N, E, D = 4, 10000, 320000, 128
P, G, R, B = 128, 128, 128, 1024

NP = 10240
EP = 327680
NPAD = NP - N
NW = 32
EPW = EP // NW
CH = 128
NCH = EPW // CH
RT = NP // 16

RB = 256
NRT = NP // RB

@functools.cache
def _mesh():
    return plsc.VectorSubcoreMesh(core_axis_name="c", subcore_axis_name="s",
                                  num_cores=2, num_subcores=16)
_PREC = lax.Precision.DEFAULT


def _dot(a, b):
    return lax.dot_general(a, b, (((1,), (0,)), ((), ())),
                           precision=_PREC, preferred_element_type=jnp.float32)


DEGW = T * NP // 16
DEGC = T * EP // CH // NW


def _deg_body(dst_hbm, out_hbm, acc, i0, i1, ones_v, zero_v, s0, s1):
    c = lax.axis_index("c")
    s = lax.axis_index("s")
    wid = s * 2 + c
    row0 = s * DEGW
    b0 = wid * DEGC * CH

    @pl.loop(0, CH // 16)
    def _init_ones(i):
        ones_v[pl.ds(i * 16, 16)] = jnp.ones((16,), jnp.float32)

    @pl.loop(0, DEGW // 16)
    def _init_zero(i):
        zero_v[pl.ds(i * 16, 16)] = jnp.zeros((16,), jnp.float32)

    pltpu.sync_copy(zero_v, acc.at[pl.ds(row0, DEGW)])
    plsc.subcore_barrier()

    pltpu.async_copy(dst_hbm.at[pl.ds(b0, CH)], i0, s0)

    @pl.loop(0, DEGC // 2)
    def _scatter(jj):
        j0 = b0 + 2 * jj * CH
        pltpu.async_copy(dst_hbm.at[pl.ds(j0 + CH, CH)], i1, s1)
        pltpu.make_async_copy(dst_hbm.at[pl.ds(j0, CH)], i0, s0).wait()
        pltpu.sync_copy(ones_v, acc.at[i0], add=True)

        @pl.when(jj < DEGC // 2 - 1)
        def _pref():
            pltpu.async_copy(dst_hbm.at[pl.ds(j0 + 2 * CH, CH)], i0, s0)

        pltpu.make_async_copy(dst_hbm.at[pl.ds(j0 + CH, CH)], i1, s1).wait()
        pltpu.sync_copy(ones_v, acc.at[i1], add=True)

    plsc.subcore_barrier()
    pltpu.sync_copy(acc.at[pl.ds(row0, DEGW)], out_hbm.at[c, pl.ds(row0, DEGW)])


@functools.cache
def _deg_call():
    return pl.kernel(
        _deg_body,
        out_type=jax.ShapeDtypeStruct((2, T * NP), jnp.float32),
        mesh=_mesh(),
        scratch_types=[
            pltpu.VMEM_SHARED((T * NP,), jnp.float32),
            pltpu.VMEM((CH,), jnp.int32),
            pltpu.VMEM((CH,), jnp.int32),
            pltpu.VMEM((CH,), jnp.float32),
            pltpu.VMEM((DEGW,), jnp.float32),
            pltpu.SemaphoreType.DMA,
            pltpu.SemaphoreType.DMA,
        ],
    )


NCHT = EP // CH


def _conv_body(t, y_hbm, ei_hbm, out_hbm, acc, i0, i1, r0, r1, s0, s1):
    c = lax.axis_index("c")
    s = lax.axis_index("s")
    wid = s * 2 + c
    row0 = s * RT

    @pl.loop(0, CH * D // 16)
    def _init_zero(i):
        r0[i // (D // 16), pl.ds((i % (D // 16)) * 16, 16)] = (
            jnp.zeros((16,), jnp.float32))

    for k in range(RT // CH):
        pltpu.async_copy(r0, acc.at[pl.ds(row0 + k * CH, CH)], s0)
    for k in range(RT // CH):
        pltpu.make_async_copy(r0, acc.at[pl.ds(row0 + k * CH, CH)],
                              s0).wait()

    plsc.subcore_barrier()

    c0 = t * NCHT + wid * NCH
    pltpu.sync_copy(ei_hbm.at[c0], i0)
    pltpu.async_copy(y_hbm.at[i0.at[0]], r0, s0)

    @pl.loop(0, NCH // 2)
    def _pairs(jj):
        j = c0 + 2 * jj
        pltpu.sync_copy(ei_hbm.at[j + 1], i1)
        pltpu.async_copy(y_hbm.at[i1.at[0]], r1, s1)
        pltpu.make_async_copy(y_hbm.at[i0.at[0]], r0, s0).wait()
        pltpu.sync_copy(r0, acc.at[i0.at[1]], add=True)

        @pl.when(jj < NCH // 2 - 1)
        def _pref():
            pltpu.sync_copy(ei_hbm.at[j + 2], i0)
            pltpu.async_copy(y_hbm.at[i0.at[0]], r0, s0)

        pltpu.make_async_copy(y_hbm.at[i1.at[0]], r1, s1).wait()
        pltpu.sync_copy(r1, acc.at[i1.at[1]], add=True)

    plsc.subcore_barrier()

    for k in range(RT // CH):
        r = row0 + k * CH
        pltpu.async_copy(acc.at[pl.ds(r, CH)], out_hbm.at[c, pl.ds(r, CH)], s1)
    for k in range(RT // CH):
        r = row0 + k * CH
        pltpu.make_async_copy(acc.at[pl.ds(r, CH)],
                              out_hbm.at[c, pl.ds(r, CH)], s1).wait()


@functools.cache
def _conv_call(t):
    return pl.kernel(
        functools.partial(_conv_body, t),
        out_type=jax.ShapeDtypeStruct((2, NP, D), jnp.float32),
        mesh=_mesh(),
        scratch_types=[
            pltpu.VMEM_SHARED((NP, D), jnp.float32),
            pltpu.VMEM((2, CH), jnp.int32),
            pltpu.VMEM((2, CH), jnp.int32),
            pltpu.VMEM((CH, D), jnp.float32),
            pltpu.VMEM((CH, D), jnp.float32),
            pltpu.SemaphoreType.DMA,
            pltpu.SemaphoreType.DMA,
        ],
    )

BW = B // NW


def _tgt_gather_body(h10, h11, h12, h13, h20, h21, h22, h23, tgt_hbm,
                     out_hbm, idx_v, rows, sem):
    c = lax.axis_index("c")
    s = lax.axis_index("s")
    wid = s * 2 + c
    b0 = wid * BW
    pltpu.sync_copy(tgt_hbm.at[pl.ds(b0, BW)], idx_v)
    planes = [h10, h20, h11, h21, h12, h22, h13, h23]
    for p, h in enumerate(planes):
        pltpu.async_copy(h.at[idx_v], rows, sem).wait()
        pltpu.sync_copy(rows, out_hbm.at[p, pl.ds(b0, BW)])


@functools.cache
def _tgt_gather_call():
    return pl.kernel(
        _tgt_gather_body,
        out_type=jax.ShapeDtypeStruct((2 * T, B, G), jnp.float32),
        mesh=_mesh(),
        scratch_types=[
            pltpu.VMEM((BW,), jnp.int32),
            pltpu.VMEM((BW, G), jnp.float32),
            pltpu.SemaphoreType.DMA,
        ],
    )


def _proj_body(x_ref, degp_ref, wp_ref, bp_ref, wg1_ref, y1_ref):
    x = x_ref[0]
    h = jnp.maximum(_dot(x, wp_ref[...]) + bp_ref[...], 0.0)
    xw = _dot(h, wg1_ref[...])
    deg = 1.0 + degp_ref[0, :] + degp_ref[1, :]
    dinv = lax.rsqrt(deg)[:, None]
    y1_ref[...] = xw * dinv


def _proj_call(t, xp, degp, wp, bp, wg1):
    return pl.pallas_call(
        _proj_body,
        grid=(NRT,),
        in_specs=[
            pl.BlockSpec((1, RB, D), lambda r, _t=t: (_t, r, 0)),
            pl.BlockSpec((2, RB), lambda r, _t=t: (0, _t * NRT + r)),
            pl.BlockSpec((D, P), lambda r: (0, 0)),
            pl.BlockSpec((1, P), lambda r: (0, 0)),
            pl.BlockSpec((P, G), lambda r: (0, 0)),
        ],
        out_specs=pl.BlockSpec((RB, G), lambda r: (r, 0)),
        out_shape=jax.ShapeDtypeStruct((NP, G), jnp.float32),
    )(xp, degp, wp, bp, wg1)


def _mid_body(p_ref, y1_ref, degp_ref, bg1_ref, wg2_ref, h1_ref, y2_ref):
    ssum = p_ref[0] + p_ref[1] + y1_ref[...]
    deg = 1.0 + degp_ref[0, :] + degp_ref[1, :]
    dinv = lax.rsqrt(deg)[:, None]
    h1 = jnp.maximum(ssum * dinv + bg1_ref[...], 0.0)
    h1_ref[...] = h1
    y2_ref[...] = _dot(h1, wg2_ref[...]) * dinv


def _mid_call(t, p, y1, degp, bg1, wg2):
    return pl.pallas_call(
        _mid_body,
        grid=(NRT,),
        in_specs=[
            pl.BlockSpec((2, RB, G), lambda r: (0, r, 0)),
            pl.BlockSpec((RB, G), lambda r: (r, 0)),
            pl.BlockSpec((2, RB), lambda r, _t=t: (0, _t * NRT + r)),
            pl.BlockSpec((1, G), lambda r: (0, 0)),
            pl.BlockSpec((G, G), lambda r: (0, 0)),
        ],
        out_specs=[
            pl.BlockSpec((RB, G), lambda r: (r, 0)),
            pl.BlockSpec((RB, G), lambda r: (r, 0)),
        ],
        out_shape=[
            jax.ShapeDtypeStruct((NP, G), jnp.float32),
            jax.ShapeDtypeStruct((NP, G), jnp.float32),
        ],
    )(p, y1, degp, bg1, wg2)


def _post_body(q_ref, y2_ref, degp_ref, bg2_ref, h2_ref):
    ssum = q_ref[0] + q_ref[1] + y2_ref[...]
    deg = 1.0 + degp_ref[0, :] + degp_ref[1, :]
    dinv = lax.rsqrt(deg)[:, None]
    h2_ref[...] = jnp.maximum(ssum * dinv + bg2_ref[...], 0.0)


def _post_call(t, q, y2, degp, bg2):
    return pl.pallas_call(
        _post_body,
        grid=(NRT,),
        in_specs=[
            pl.BlockSpec((2, RB, G), lambda r: (0, r, 0)),
            pl.BlockSpec((RB, G), lambda r: (r, 0)),
            pl.BlockSpec((2, RB), lambda r, _t=t: (0, _t * NRT + r)),
            pl.BlockSpec((1, G), lambda r: (0, 0)),
        ],
        out_specs=pl.BlockSpec((RB, G), lambda r: (r, 0)),
        out_shape=jax.ShapeDtypeStruct((NP, G), jnp.float32),
    )(q, y2, degp, bg2)


RBG = 256


def _gru_body(g_ref, wih_ref, whh_ref, bih_ref, bhh_ref, watt_ref,
              batt_ref, wp1_ref, bp1_ref, wp2_ref, bp2_ref, o_ref):
    h = jnp.zeros((RBG, R), jnp.float32)
    wih = wih_ref[...]
    whh = whh_ref[...]
    hs, es = [], []
    for t in range(T):
        xt = jnp.concatenate([g_ref[2 * t], g_ref[2 * t + 1]], axis=1)
        gi = _dot(xt, wih) + bih_ref[...]
        gh = _dot(h, whh) + bhh_ref[...]
        rg = jax.nn.sigmoid(gi[:, :R] + gh[:, :R])
        zg = jax.nn.sigmoid(gi[:, R:2 * R] + gh[:, R:2 * R])
        ng = jnp.tanh(gi[:, 2 * R:] + rg * gh[:, 2 * R:])
        h = (1.0 - zg) * ng + zg * h
        hs.append(h)
        es.append(jnp.tanh(_dot(h, watt_ref[...]) + batt_ref[0, 0]))
    m = jnp.maximum(jnp.maximum(es[0], es[1]), jnp.maximum(es[2], es[3]))
    ws = [jnp.exp(s - m) for s in es]
    den = ws[0] + ws[1] + ws[2] + ws[3]
    rep = (hs[0] * ws[0] + hs[1] * ws[1] + hs[2] * ws[2] + hs[3] * ws[3]) / den
    hp = jnp.maximum(_dot(rep, wp1_ref[...]) + bp1_ref[...], 0.0)
    o_ref[...] = _dot(hp, wp2_ref[...]) + bp2_ref[0, 0]


def _gru_call(g, wih, whh, bih, bhh, watt, batt, wp1, bp1, wp2, bp2):
    return pl.pallas_call(
        _gru_body,
        grid=(B // RBG,),
        in_specs=[
            pl.BlockSpec((2 * T, RBG, G), lambda r: (0, r, 0)),
            pl.BlockSpec((2 * G, 3 * R), lambda r: (0, 0)),
            pl.BlockSpec((R, 3 * R), lambda r: (0, 0)),
            pl.BlockSpec((1, 3 * R), lambda r: (0, 0)),
            pl.BlockSpec((1, 3 * R), lambda r: (0, 0)),
            pl.BlockSpec((R, 1), lambda r: (0, 0)),
            pl.BlockSpec((1, 1), lambda r: (0, 0)),
            pl.BlockSpec((R, 16), lambda r: (0, 0)),
            pl.BlockSpec((1, 16), lambda r: (0, 0)),
            pl.BlockSpec((16, 1), lambda r: (0, 0)),
            pl.BlockSpec((1, 1), lambda r: (0, 0)),
        ],
        out_specs=pl.BlockSpec((RBG, 1), lambda r: (r, 0)),
        out_shape=jax.ShapeDtypeStruct((B, 1), jnp.float32),
    )(g, wih, whh, bih, bhh, watt, batt, wp1, bp1, wp2, bp2)


def kernel(x_seq, edge_index_seq, target_indices, W_proj, b_proj, W_g1, b_g1,
           W_g2, b_g2, W_ih, W_hh, b_ih, b_hh, W_att, b_att, W_p1, b_p1,
           W_p2, b_p2):
    xp = jnp.pad(x_seq, ((0, 0), (0, NP - N), (0, 0)))

    src = edge_index_seq[:, 0, :]
    dst = edge_index_seq[:, 1, :]
    padi = (N + (jnp.arange(EP - E, dtype=jnp.int32) % NPAD))[None, :]
    pads = jnp.broadcast_to(padi, (T, EP - E))
    srcp = jnp.concatenate([src, pads], axis=1)
    dstp = jnp.concatenate([dst, pads], axis=1)
    ei = jnp.stack([srcp.reshape(T, NCHT, CH), dstp.reshape(T, NCHT, CH)],
                   axis=2).reshape(T * NCHT, 2, CH)

    dst_off = (dstp + (jnp.arange(T, dtype=jnp.int32) * NP)[:, None])
    degp = _deg_call()(dst_off.reshape(-1))

    bp = b_proj.reshape(1, P)
    bg1 = b_g1.reshape(1, G)
    bg2 = b_g2.reshape(1, G)

    h1s, h2s = [], []
    for t in range(T):
        y1 = _proj_call(t, xp, degp, W_proj, bp, W_g1)
        p = _conv_call(t)(y1, ei)
        h1, y2 = _mid_call(t, p, y1, degp, bg1, W_g2)
        q = _conv_call(t)(y2, ei)
        h2 = _post_call(t, q, y2, degp, bg2)
        h1s.append(h1)
        h2s.append(h2)

    g = _tgt_gather_call()(h1s[0], h1s[1], h1s[2], h1s[3],
                           h2s[0], h2s[1], h2s[2], h2s[3], target_indices)

    out = _gru_call(
        g, W_ih.T, W_hh.T,
        b_ih.reshape(1, 3 * R), b_hh.reshape(1, 3 * R),
        W_att, b_att.reshape(1, 1),
        W_p1, b_p1.reshape(1, 16), W_p2, b_p2.reshape(1, 1))
    return out


# speedup vs baseline: 26.7413x; 1.1093x over previous
import functools

import jax
import jax.numpy as jnp
from jax import lax
from jax.experimental import pallas as pl
from jax.experimental.pallas import tpu as pltpu
from jax.experimental.pallas import tpu_sc as plsc

T, N, E, D = 4, 10000, 320000, 128
P, G, R, B = 128, 128, 128, 1024

NP = 10240
EP = 327680
NPAD = NP - N
NW = 32
EPW = EP // NW
CH = 128
NCH = EPW // CH
RT = NP // 16

RB = 256
NRT = NP // RB

@functools.cache
def _mesh():
    return plsc.VectorSubcoreMesh(core_axis_name="c", subcore_axis_name="s",
                                  num_cores=2, num_subcores=16)
_PREC = lax.Precision.DEFAULT


def _dot(a, b):
    return lax.dot_general(a, b, (((1,), (0,)), ((), ())),
                           precision=_PREC, preferred_element_type=jnp.float32)


DEGW = T * NP // 16
DEGC = T * EP // CH // NW


def _deg_body(dst_hbm, out_hbm, acc, i0, i1, ones_v, zero_v, s0, s1):
    c = lax.axis_index("c")
    s = lax.axis_index("s")
    wid = s * 2 + c
    row0 = s * DEGW
    b0 = wid * DEGC * CH

    @pl.loop(0, CH // 16)
    def _init_ones(i):
        ones_v[pl.ds(i * 16, 16)] = jnp.ones((16,), jnp.float32)

    @pl.loop(0, DEGW // 16)
    def _init_zero(i):
        zero_v[pl.ds(i * 16, 16)] = jnp.zeros((16,), jnp.float32)

    pltpu.sync_copy(zero_v, acc.at[pl.ds(row0, DEGW)])
    plsc.subcore_barrier()

    pltpu.async_copy(dst_hbm.at[pl.ds(b0, CH)], i0, s0)

    @pl.loop(0, DEGC // 2)
    def _scatter(jj):
        j0 = b0 + 2 * jj * CH
        pltpu.async_copy(dst_hbm.at[pl.ds(j0 + CH, CH)], i1, s1)
        pltpu.make_async_copy(dst_hbm.at[pl.ds(j0, CH)], i0, s0).wait()
        pltpu.sync_copy(ones_v, acc.at[i0], add=True)

        @pl.when(jj < DEGC // 2 - 1)
        def _pref():
            pltpu.async_copy(dst_hbm.at[pl.ds(j0 + 2 * CH, CH)], i0, s0)

        pltpu.make_async_copy(dst_hbm.at[pl.ds(j0 + CH, CH)], i1, s1).wait()
        pltpu.sync_copy(ones_v, acc.at[i1], add=True)

    plsc.subcore_barrier()
    pltpu.sync_copy(acc.at[pl.ds(row0, DEGW)], out_hbm.at[c, pl.ds(row0, DEGW)])


@functools.cache
def _deg_call():
    return pl.kernel(
        _deg_body,
        out_type=jax.ShapeDtypeStruct((2, T * NP), jnp.float32),
        mesh=_mesh(),
        scratch_types=[
            pltpu.VMEM_SHARED((T * NP,), jnp.float32),
            pltpu.VMEM((CH,), jnp.int32),
            pltpu.VMEM((CH,), jnp.int32),
            pltpu.VMEM((CH,), jnp.float32),
            pltpu.VMEM((DEGW,), jnp.float32),
            pltpu.SemaphoreType.DMA,
            pltpu.SemaphoreType.DMA,
        ],
    )


NCHT = EP // CH


def _conv_body(t, y_hbm, ei_hbm, out_hbm, acc, ib0, ib1, ib2, ib3, rb0, rb1,
               si0, si1, si2, si3, sg0, sg1, ss0, ss1):
    ibs = (ib0, ib1, ib2, ib3)
    rbs = (rb0, rb1)
    sis = (si0, si1, si2, si3)
    sgs = (sg0, sg1)
    sss = (ss0, ss1)
    c = lax.axis_index("c")
    s = lax.axis_index("s")
    wid = s * 2 + c
    row0 = s * RT

    @pl.loop(0, CH * D // 16)
    def _init_zero(i):
        rb0[i // (D // 16), pl.ds((i % (D // 16)) * 16, 16)] = (
            jnp.zeros((16,), jnp.float32))

    for k in range(RT // CH):
        pltpu.async_copy(rb0, acc.at[pl.ds(row0 + k * CH, CH)], sg0)
    for k in range(RT // CH):
        pltpu.make_async_copy(rb0, acc.at[pl.ds(row0 + k * CH, CH)],
                              sg0).wait()

    plsc.subcore_barrier()

    c0 = t * NCHT + wid * NCH
    for k in range(3):
        pltpu.async_copy(ei_hbm.at[c0 + k], ibs[k], sis[k])
    pltpu.make_async_copy(ei_hbm.at[c0], ibs[0], sis[0]).wait()
    pltpu.async_copy(y_hbm.at[ib0.at[0]], rb0, sg0)

    @pl.loop(0, NCH // 4)
    def _quad(q):
        for k in range(4):
            rel = 4 * q + k
            j = c0 + rel
            kp1, kp3 = (k + 1) % 4, (k + 3) % 4
            b, bp1 = k % 2, (k + 1) % 2

            @pl.when(rel > 0)
            def _wait_prev_scatter():
                pltpu.make_async_copy(rbs[bp1], acc.at[ibs[kp3].at[1]],
                                      sss[bp1]).wait()

            @pl.when(rel + 3 < NCH)
            def _fetch_idx():
                pltpu.async_copy(ei_hbm.at[j + 3], ibs[kp3], sis[kp3])

            @pl.when(rel + 1 < NCH)
            def _issue_gather():
                pltpu.make_async_copy(ei_hbm.at[j + 1], ibs[kp1],
                                      sis[kp1]).wait()
                pltpu.async_copy(y_hbm.at[ibs[kp1].at[0]], rbs[bp1], sgs[bp1])

            pltpu.make_async_copy(y_hbm.at[ibs[k].at[0]], rbs[b],
                                  sgs[b]).wait()
            pltpu.async_copy(rbs[b], acc.at[ibs[k].at[1]], sss[b], add=True)

    pltpu.make_async_copy(rbs[1], acc.at[ibs[3].at[1]], sss[1]).wait()

    plsc.subcore_barrier()

    for k in range(RT // CH):
        r = row0 + k * CH
        pltpu.async_copy(acc.at[pl.ds(r, CH)], out_hbm.at[c, pl.ds(r, CH)],
                         sg1)
    for k in range(RT // CH):
        r = row0 + k * CH
        pltpu.make_async_copy(acc.at[pl.ds(r, CH)],
                              out_hbm.at[c, pl.ds(r, CH)], sg1).wait()


@functools.cache
def _conv_call(t):
    return pl.kernel(
        functools.partial(_conv_body, t),
        out_type=jax.ShapeDtypeStruct((2, NP, D), jnp.float32),
        mesh=_mesh(),
        scratch_types=[
            pltpu.VMEM_SHARED((NP, D), jnp.float32),
            pltpu.VMEM((2, CH), jnp.int32),
            pltpu.VMEM((2, CH), jnp.int32),
            pltpu.VMEM((2, CH), jnp.int32),
            pltpu.VMEM((2, CH), jnp.int32),
            pltpu.VMEM((CH, D), jnp.float32),
            pltpu.VMEM((CH, D), jnp.float32),
            pltpu.SemaphoreType.DMA,
            pltpu.SemaphoreType.DMA,
            pltpu.SemaphoreType.DMA,
            pltpu.SemaphoreType.DMA,
            pltpu.SemaphoreType.DMA,
            pltpu.SemaphoreType.DMA,
            pltpu.SemaphoreType.DMA,
            pltpu.SemaphoreType.DMA,
        ],
    )

BW = B // NW


def _tgt_gather_body(h10, h11, h12, h13, h20, h21, h22, h23, tgt_hbm,
                     out_hbm, idx_v, rows, sem):
    c = lax.axis_index("c")
    s = lax.axis_index("s")
    wid = s * 2 + c
    b0 = wid * BW
    pltpu.sync_copy(tgt_hbm.at[pl.ds(b0, BW)], idx_v)
    planes = [h10, h20, h11, h21, h12, h22, h13, h23]
    for p, h in enumerate(planes):
        pltpu.async_copy(h.at[idx_v], rows, sem).wait()
        pltpu.sync_copy(rows, out_hbm.at[p, pl.ds(b0, BW)])


@functools.cache
def _tgt_gather_call():
    return pl.kernel(
        _tgt_gather_body,
        out_type=jax.ShapeDtypeStruct((2 * T, B, G), jnp.float32),
        mesh=_mesh(),
        scratch_types=[
            pltpu.VMEM((BW,), jnp.int32),
            pltpu.VMEM((BW, G), jnp.float32),
            pltpu.SemaphoreType.DMA,
        ],
    )


def _proj_body(x_ref, degp_ref, wp_ref, bp_ref, wg1_ref, y1_ref):
    x = x_ref[0]
    h = jnp.maximum(_dot(x, wp_ref[...]) + bp_ref[...], 0.0)
    xw = _dot(h, wg1_ref[...])
    deg = 1.0 + degp_ref[0, :] + degp_ref[1, :]
    dinv = lax.rsqrt(deg)[:, None]
    y1_ref[...] = xw * dinv


def _proj_call(t, xp, degp, wp, bp, wg1):
    return pl.pallas_call(
        _proj_body,
        grid=(NRT,),
        in_specs=[
            pl.BlockSpec((1, RB, D), lambda r, _t=t: (_t, r, 0)),
            pl.BlockSpec((2, RB), lambda r, _t=t: (0, _t * NRT + r)),
            pl.BlockSpec((D, P), lambda r: (0, 0)),
            pl.BlockSpec((1, P), lambda r: (0, 0)),
            pl.BlockSpec((P, G), lambda r: (0, 0)),
        ],
        out_specs=pl.BlockSpec((RB, G), lambda r: (r, 0)),
        out_shape=jax.ShapeDtypeStruct((NP, G), jnp.float32),
    )(xp, degp, wp, bp, wg1)


def _mid_body(p_ref, y1_ref, degp_ref, bg1_ref, wg2_ref, h1_ref, y2_ref):
    ssum = p_ref[0] + p_ref[1] + y1_ref[...]
    deg = 1.0 + degp_ref[0, :] + degp_ref[1, :]
    dinv = lax.rsqrt(deg)[:, None]
    h1 = jnp.maximum(ssum * dinv + bg1_ref[...], 0.0)
    h1_ref[...] = h1
    y2_ref[...] = _dot(h1, wg2_ref[...]) * dinv


def _mid_call(t, p, y1, degp, bg1, wg2):
    return pl.pallas_call(
        _mid_body,
        grid=(NRT,),
        in_specs=[
            pl.BlockSpec((2, RB, G), lambda r: (0, r, 0)),
            pl.BlockSpec((RB, G), lambda r: (r, 0)),
            pl.BlockSpec((2, RB), lambda r, _t=t: (0, _t * NRT + r)),
            pl.BlockSpec((1, G), lambda r: (0, 0)),
            pl.BlockSpec((G, G), lambda r: (0, 0)),
        ],
        out_specs=[
            pl.BlockSpec((RB, G), lambda r: (r, 0)),
            pl.BlockSpec((RB, G), lambda r: (r, 0)),
        ],
        out_shape=[
            jax.ShapeDtypeStruct((NP, G), jnp.float32),
            jax.ShapeDtypeStruct((NP, G), jnp.float32),
        ],
    )(p, y1, degp, bg1, wg2)


def _post_body(q_ref, y2_ref, degp_ref, bg2_ref, h2_ref):
    ssum = q_ref[0] + q_ref[1] + y2_ref[...]
    deg = 1.0 + degp_ref[0, :] + degp_ref[1, :]
    dinv = lax.rsqrt(deg)[:, None]
    h2_ref[...] = jnp.maximum(ssum * dinv + bg2_ref[...], 0.0)


def _post_call(t, q, y2, degp, bg2):
    return pl.pallas_call(
        _post_body,
        grid=(NRT,),
        in_specs=[
            pl.BlockSpec((2, RB, G), lambda r: (0, r, 0)),
            pl.BlockSpec((RB, G), lambda r: (r, 0)),
            pl.BlockSpec((2, RB), lambda r, _t=t: (0, _t * NRT + r)),
            pl.BlockSpec((1, G), lambda r: (0, 0)),
        ],
        out_specs=pl.BlockSpec((RB, G), lambda r: (r, 0)),
        out_shape=jax.ShapeDtypeStruct((NP, G), jnp.float32),
    )(q, y2, degp, bg2)


RBG = 256


def _gru_body(g_ref, wih_ref, whh_ref, bih_ref, bhh_ref, watt_ref,
              batt_ref, wp1_ref, bp1_ref, wp2_ref, bp2_ref, o_ref):
    h = jnp.zeros((RBG, R), jnp.float32)
    wih = wih_ref[...]
    whh = whh_ref[...]
    hs, es = [], []
    for t in range(T):
        xt = jnp.concatenate([g_ref[2 * t], g_ref[2 * t + 1]], axis=1)
        gi = _dot(xt, wih) + bih_ref[...]
        gh = _dot(h, whh) + bhh_ref[...]
        rg = jax.nn.sigmoid(gi[:, :R] + gh[:, :R])
        zg = jax.nn.sigmoid(gi[:, R:2 * R] + gh[:, R:2 * R])
        ng = jnp.tanh(gi[:, 2 * R:] + rg * gh[:, 2 * R:])
        h = (1.0 - zg) * ng + zg * h
        hs.append(h)
        es.append(jnp.tanh(_dot(h, watt_ref[...]) + batt_ref[0, 0]))
    m = jnp.maximum(jnp.maximum(es[0], es[1]), jnp.maximum(es[2], es[3]))
    ws = [jnp.exp(s - m) for s in es]
    den = ws[0] + ws[1] + ws[2] + ws[3]
    rep = (hs[0] * ws[0] + hs[1] * ws[1] + hs[2] * ws[2] + hs[3] * ws[3]) / den
    hp = jnp.maximum(_dot(rep, wp1_ref[...]) + bp1_ref[...], 0.0)
    o_ref[...] = _dot(hp, wp2_ref[...]) + bp2_ref[0, 0]


def _gru_call(g, wih, whh, bih, bhh, watt, batt, wp1, bp1, wp2, bp2):
    return pl.pallas_call(
        _gru_body,
        grid=(B // RBG,),
        in_specs=[
            pl.BlockSpec((2 * T, RBG, G), lambda r: (0, r, 0)),
            pl.BlockSpec((2 * G, 3 * R), lambda r: (0, 0)),
            pl.BlockSpec((R, 3 * R), lambda r: (0, 0)),
            pl.BlockSpec((1, 3 * R), lambda r: (0, 0)),
            pl.BlockSpec((1, 3 * R), lambda r: (0, 0)),
            pl.BlockSpec((R, 1), lambda r: (0, 0)),
            pl.BlockSpec((1, 1), lambda r: (0, 0)),
            pl.BlockSpec((R, 16), lambda r: (0, 0)),
            pl.BlockSpec((1, 16), lambda r: (0, 0)),
            pl.BlockSpec((16, 1), lambda r: (0, 0)),
            pl.BlockSpec((1, 1), lambda r: (0, 0)),
        ],
        out_specs=pl.BlockSpec((RBG, 1), lambda r: (r, 0)),
        out_shape=jax.ShapeDtypeStruct((B, 1), jnp.float32),
    )(g, wih, whh, bih, bhh, watt, batt, wp1, bp1, wp2, bp2)


def kernel(x_seq, edge_index_seq, target_indices, W_proj, b_proj, W_g1, b_g1,
           W_g2, b_g2, W_ih, W_hh, b_ih, b_hh, W_att, b_att, W_p1, b_p1,
           W_p2, b_p2):
    xp = jnp.pad(x_seq, ((0, 0), (0, NP - N), (0, 0)))

    src = edge_index_seq[:, 0, :]
    dst = edge_index_seq[:, 1, :]
    padi = (N + (jnp.arange(EP - E, dtype=jnp.int32) % NPAD))[None, :]
    pads = jnp.broadcast_to(padi, (T, EP - E))
    srcp = jnp.concatenate([src, pads], axis=1)
    dstp = jnp.concatenate([dst, pads], axis=1)
    ei = jnp.stack([srcp.reshape(T, NCHT, CH), dstp.reshape(T, NCHT, CH)],
                   axis=2).reshape(T * NCHT, 2, CH)

    dst_off = (dstp + (jnp.arange(T, dtype=jnp.int32) * NP)[:, None])
    degp = _deg_call()(dst_off.reshape(-1))

    bp = b_proj.reshape(1, P)
    bg1 = b_g1.reshape(1, G)
    bg2 = b_g2.reshape(1, G)

    h1s, h2s = [], []
    for t in range(T):
        y1 = _proj_call(t, xp, degp, W_proj, bp, W_g1)
        p = _conv_call(t)(y1, ei)
        h1, y2 = _mid_call(t, p, y1, degp, bg1, W_g2)
        q = _conv_call(t)(y2, ei)
        h2 = _post_call(t, q, y2, degp, bg2)
        h1s.append(h1)
        h2s.append(h2)

    g = _tgt_gather_call()(h1s[0], h1s[1], h1s[2], h1s[3],
                           h2s[0], h2s[1], h2s[2], h2s[3], target_indices)

    out = _gru_call(
        g, W_ih.T, W_hh.T,
        b_ih.reshape(1, 3 * R), b_hh.reshape(1, 3 * R),
        W_att, b_att.reshape(1, 1),
        W_p1, b_p1.reshape(1, 16), W_p2, b_p2.reshape(1, 1))
    return out


# speedup vs baseline: 27.7818x; 1.0389x over previous
import functools

import jax
import jax.numpy as jnp
from jax import lax
from jax.experimental import pallas as pl
from jax.experimental.pallas import tpu as pltpu
from jax.experimental.pallas import tpu_sc as plsc

T, N, E, D = 4, 10000, 320000, 128
P, G, R, B = 128, 128, 128, 1024

NP = 10240
EP = 327680
NPAD = NP - N
NW = 32
EPW = EP // NW
CH = 128
NCH = EPW // CH
RT = NP // 16

RB = 256
NRT = NP // RB

@functools.cache
def _mesh():
    return plsc.VectorSubcoreMesh(core_axis_name="c", subcore_axis_name="s",
                                  num_cores=2, num_subcores=16)
_PREC = lax.Precision.DEFAULT


def _dot(a, b):
    return lax.dot_general(a, b, (((1,), (0,)), ((), ())),
                           precision=_PREC, preferred_element_type=jnp.float32)


DEGW = T * NP // 16
DEGC = T * EP // CH // NW


SLC = 8
NSL = T * EP // CH // NW // SLC


def _deg_body(dst_hbm, out_hbm, acc, a0, a1, ones_v, zero_v, sa0, sa1, sc):
    c = lax.axis_index("c")
    s = lax.axis_index("s")
    wid = s * 2 + c
    row0 = s * DEGW
    b0 = wid * DEGC

    @pl.loop(0, CH // 16)
    def _init_ones(i):
        ones_v[pl.ds(i * 16, 16)] = jnp.ones((16,), jnp.float32)

    @pl.loop(0, DEGW // 16)
    def _init_zero(i):
        zero_v[pl.ds(i * 16, 16)] = jnp.zeros((16,), jnp.float32)

    pltpu.sync_copy(zero_v, acc.at[pl.ds(row0, DEGW)])
    plsc.subcore_barrier()

    bufs = (a0, a1)
    sems = (sa0, sa1)
    pltpu.async_copy(dst_hbm.at[pl.ds(b0, SLC)], a0, sa0)

    @pl.loop(0, NSL // 2)
    def _slabs(ss_):
        for b in range(2):
            sl = 2 * ss_ + b
            buf, sem = bufs[b], sems[b]
            base = b0 + sl * SLC
            pltpu.make_async_copy(dst_hbm.at[pl.ds(base, SLC)], buf,
                                  sem).wait()

            @pl.when(sl + 1 < NSL)
            def _pref():
                pltpu.async_copy(dst_hbm.at[pl.ds(base + SLC, SLC)],
                                 bufs[1 - b], sems[1 - b])

            for k in range(SLC):
                pltpu.async_copy(ones_v, acc.at[buf.at[k]], sc, add=True)
            for k in range(SLC):
                pltpu.make_async_copy(ones_v, acc.at[buf.at[k]], sc).wait()

    plsc.subcore_barrier()
    pltpu.sync_copy(acc.at[pl.ds(row0, DEGW)], out_hbm.at[c, pl.ds(row0, DEGW)])


@functools.cache
def _deg_call():
    return pl.kernel(
        _deg_body,
        out_type=jax.ShapeDtypeStruct((2, T * NP), jnp.float32),
        mesh=_mesh(),
        scratch_types=[
            pltpu.VMEM_SHARED((T * NP,), jnp.float32),
            pltpu.VMEM((SLC, CH), jnp.int32),
            pltpu.VMEM((SLC, CH), jnp.int32),
            pltpu.VMEM((CH,), jnp.float32),
            pltpu.VMEM((DEGW,), jnp.float32),
            pltpu.SemaphoreType.DMA,
            pltpu.SemaphoreType.DMA,
            pltpu.SemaphoreType.DMA,
        ],
    )


NCHT = EP // CH


def _conv_body(t, y_hbm, ei_hbm, out_hbm, acc, ib0, ib1, ib2, ib3, rb0, rb1,
               si0, si1, si2, si3, sg0, sg1, ss0, ss1):
    ibs = (ib0, ib1, ib2, ib3)
    rbs = (rb0, rb1)
    sis = (si0, si1, si2, si3)
    sgs = (sg0, sg1)
    sss = (ss0, ss1)
    c = lax.axis_index("c")
    s = lax.axis_index("s")
    wid = s * 2 + c
    row0 = s * RT

    @pl.loop(0, CH * D // 16)
    def _init_zero(i):
        rb0[i // (D // 16), pl.ds((i % (D // 16)) * 16, 16)] = (
            jnp.zeros((16,), jnp.float32))

    for k in range(RT // CH):
        pltpu.async_copy(rb0, acc.at[pl.ds(row0 + k * CH, CH)], sg0)
    for k in range(RT // CH):
        pltpu.make_async_copy(rb0, acc.at[pl.ds(row0 + k * CH, CH)],
                              sg0).wait()

    plsc.subcore_barrier()

    c0 = t * NCHT + wid * NCH
    for k in range(3):
        pltpu.async_copy(ei_hbm.at[c0 + k], ibs[k], sis[k])
    pltpu.make_async_copy(ei_hbm.at[c0], ibs[0], sis[0]).wait()
    pltpu.async_copy(y_hbm.at[ib0.at[0]], rb0, sg0)

    @pl.loop(0, NCH // 4)
    def _quad(q):
        for k in range(4):
            rel = 4 * q + k
            j = c0 + rel
            kp1, kp3 = (k + 1) % 4, (k + 3) % 4
            b, bp1 = k % 2, (k + 1) % 2

            @pl.when(rel > 0)
            def _wait_prev_scatter():
                pltpu.make_async_copy(rbs[bp1], acc.at[ibs[kp3].at[1]],
                                      sss[bp1]).wait()

            @pl.when(rel + 3 < NCH)
            def _fetch_idx():
                pltpu.async_copy(ei_hbm.at[j + 3], ibs[kp3], sis[kp3])

            @pl.when(rel + 1 < NCH)
            def _issue_gather():
                pltpu.make_async_copy(ei_hbm.at[j + 1], ibs[kp1],
                                      sis[kp1]).wait()
                pltpu.async_copy(y_hbm.at[ibs[kp1].at[0]], rbs[bp1], sgs[bp1])

            pltpu.make_async_copy(y_hbm.at[ibs[k].at[0]], rbs[b],
                                  sgs[b]).wait()
            pltpu.async_copy(rbs[b], acc.at[ibs[k].at[1]], sss[b], add=True)

    pltpu.make_async_copy(rbs[1], acc.at[ibs[3].at[1]], sss[1]).wait()

    plsc.subcore_barrier()

    for k in range(RT // CH):
        r = row0 + k * CH
        pltpu.async_copy(acc.at[pl.ds(r, CH)], out_hbm.at[c, pl.ds(r, CH)],
                         sg1)
    for k in range(RT // CH):
        r = row0 + k * CH
        pltpu.make_async_copy(acc.at[pl.ds(r, CH)],
                              out_hbm.at[c, pl.ds(r, CH)], sg1).wait()


@functools.cache
def _conv_call(t):
    return pl.kernel(
        functools.partial(_conv_body, t),
        out_type=jax.ShapeDtypeStruct((2, NP, D), jnp.float32),
        mesh=_mesh(),
        scratch_types=[
            pltpu.VMEM_SHARED((NP, D), jnp.float32),
            pltpu.VMEM((2, CH), jnp.int32),
            pltpu.VMEM((2, CH), jnp.int32),
            pltpu.VMEM((2, CH), jnp.int32),
            pltpu.VMEM((2, CH), jnp.int32),
            pltpu.VMEM((CH, D), jnp.float32),
            pltpu.VMEM((CH, D), jnp.float32),
            pltpu.SemaphoreType.DMA,
            pltpu.SemaphoreType.DMA,
            pltpu.SemaphoreType.DMA,
            pltpu.SemaphoreType.DMA,
            pltpu.SemaphoreType.DMA,
            pltpu.SemaphoreType.DMA,
            pltpu.SemaphoreType.DMA,
            pltpu.SemaphoreType.DMA,
        ],
    )

BW = B // NW


def _tgt_gather_body(h10, h11, h12, h13, h20, h21, h22, h23, tgt_hbm,
                     out_hbm, idx_v, rows, sem):
    c = lax.axis_index("c")
    s = lax.axis_index("s")
    wid = s * 2 + c
    b0 = wid * BW
    pltpu.sync_copy(tgt_hbm.at[pl.ds(b0, BW)], idx_v)
    planes = [h10, h20, h11, h21, h12, h22, h13, h23]
    for p, h in enumerate(planes):
        pltpu.async_copy(h.at[idx_v], rows, sem).wait()
        pltpu.sync_copy(rows, out_hbm.at[p, pl.ds(b0, BW)])


@functools.cache
def _tgt_gather_call():
    return pl.kernel(
        _tgt_gather_body,
        out_type=jax.ShapeDtypeStruct((2 * T, B, G), jnp.float32),
        mesh=_mesh(),
        scratch_types=[
            pltpu.VMEM((BW,), jnp.int32),
            pltpu.VMEM((BW, G), jnp.float32),
            pltpu.SemaphoreType.DMA,
        ],
    )


def _proj_body(x_ref, degp_ref, wp_ref, bp_ref, wg1_ref, y1_ref):
    x = x_ref[0]
    h = jnp.maximum(_dot(x, wp_ref[...]) + bp_ref[...], 0.0)
    xw = _dot(h, wg1_ref[...])
    deg = 1.0 + degp_ref[0, :] + degp_ref[1, :]
    dinv = lax.rsqrt(deg)[:, None]
    y1_ref[...] = xw * dinv


def _proj_call(t, xp, degp, wp, bp, wg1):
    return pl.pallas_call(
        _proj_body,
        grid=(NRT,),
        in_specs=[
            pl.BlockSpec((1, RB, D), lambda r, _t=t: (_t, r, 0)),
            pl.BlockSpec((2, RB), lambda r, _t=t: (0, _t * NRT + r)),
            pl.BlockSpec((D, P), lambda r: (0, 0)),
            pl.BlockSpec((1, P), lambda r: (0, 0)),
            pl.BlockSpec((P, G), lambda r: (0, 0)),
        ],
        out_specs=pl.BlockSpec((RB, G), lambda r: (r, 0)),
        out_shape=jax.ShapeDtypeStruct((NP, G), jnp.float32),
    )(xp, degp, wp, bp, wg1)


def _mid_body(p_ref, y1_ref, degp_ref, bg1_ref, wg2_ref, h1_ref, y2_ref):
    ssum = p_ref[0] + p_ref[1] + y1_ref[...]
    deg = 1.0 + degp_ref[0, :] + degp_ref[1, :]
    dinv = lax.rsqrt(deg)[:, None]
    h1 = jnp.maximum(ssum * dinv + bg1_ref[...], 0.0)
    h1_ref[...] = h1
    y2_ref[...] = _dot(h1, wg2_ref[...]) * dinv


def _mid_call(t, p, y1, degp, bg1, wg2):
    return pl.pallas_call(
        _mid_body,
        grid=(NRT,),
        in_specs=[
            pl.BlockSpec((2, RB, G), lambda r: (0, r, 0)),
            pl.BlockSpec((RB, G), lambda r: (r, 0)),
            pl.BlockSpec((2, RB), lambda r, _t=t: (0, _t * NRT + r)),
            pl.BlockSpec((1, G), lambda r: (0, 0)),
            pl.BlockSpec((G, G), lambda r: (0, 0)),
        ],
        out_specs=[
            pl.BlockSpec((RB, G), lambda r: (r, 0)),
            pl.BlockSpec((RB, G), lambda r: (r, 0)),
        ],
        out_shape=[
            jax.ShapeDtypeStruct((NP, G), jnp.float32),
            jax.ShapeDtypeStruct((NP, G), jnp.float32),
        ],
    )(p, y1, degp, bg1, wg2)


def _post_body(q_ref, y2_ref, degp_ref, bg2_ref, h2_ref):
    ssum = q_ref[0] + q_ref[1] + y2_ref[...]
    deg = 1.0 + degp_ref[0, :] + degp_ref[1, :]
    dinv = lax.rsqrt(deg)[:, None]
    h2_ref[...] = jnp.maximum(ssum * dinv + bg2_ref[...], 0.0)


def _post_call(t, q, y2, degp, bg2):
    return pl.pallas_call(
        _post_body,
        grid=(NRT,),
        in_specs=[
            pl.BlockSpec((2, RB, G), lambda r: (0, r, 0)),
            pl.BlockSpec((RB, G), lambda r: (r, 0)),
            pl.BlockSpec((2, RB), lambda r, _t=t: (0, _t * NRT + r)),
            pl.BlockSpec((1, G), lambda r: (0, 0)),
        ],
        out_specs=pl.BlockSpec((RB, G), lambda r: (r, 0)),
        out_shape=jax.ShapeDtypeStruct((NP, G), jnp.float32),
    )(q, y2, degp, bg2)


RBG = 256


def _gru_body(g_ref, wih_ref, whh_ref, bih_ref, bhh_ref, watt_ref,
              batt_ref, wp1_ref, bp1_ref, wp2_ref, bp2_ref, o_ref):
    h = jnp.zeros((RBG, R), jnp.float32)
    wih = wih_ref[...]
    whh = whh_ref[...]
    hs, es = [], []
    for t in range(T):
        xt = jnp.concatenate([g_ref[2 * t], g_ref[2 * t + 1]], axis=1)
        gi = _dot(xt, wih) + bih_ref[...]
        gh = _dot(h, whh) + bhh_ref[...]
        rg = jax.nn.sigmoid(gi[:, :R] + gh[:, :R])
        zg = jax.nn.sigmoid(gi[:, R:2 * R] + gh[:, R:2 * R])
        ng = jnp.tanh(gi[:, 2 * R:] + rg * gh[:, 2 * R:])
        h = (1.0 - zg) * ng + zg * h
        hs.append(h)
        es.append(jnp.tanh(_dot(h, watt_ref[...]) + batt_ref[0, 0]))
    m = jnp.maximum(jnp.maximum(es[0], es[1]), jnp.maximum(es[2], es[3]))
    ws = [jnp.exp(s - m) for s in es]
    den = ws[0] + ws[1] + ws[2] + ws[3]
    rep = (hs[0] * ws[0] + hs[1] * ws[1] + hs[2] * ws[2] + hs[3] * ws[3]) / den
    hp = jnp.maximum(_dot(rep, wp1_ref[...]) + bp1_ref[...], 0.0)
    o_ref[...] = _dot(hp, wp2_ref[...]) + bp2_ref[0, 0]


def _gru_call(g, wih, whh, bih, bhh, watt, batt, wp1, bp1, wp2, bp2):
    return pl.pallas_call(
        _gru_body,
        grid=(B // RBG,),
        in_specs=[
            pl.BlockSpec((2 * T, RBG, G), lambda r: (0, r, 0)),
            pl.BlockSpec((2 * G, 3 * R), lambda r: (0, 0)),
            pl.BlockSpec((R, 3 * R), lambda r: (0, 0)),
            pl.BlockSpec((1, 3 * R), lambda r: (0, 0)),
            pl.BlockSpec((1, 3 * R), lambda r: (0, 0)),
            pl.BlockSpec((R, 1), lambda r: (0, 0)),
            pl.BlockSpec((1, 1), lambda r: (0, 0)),
            pl.BlockSpec((R, 16), lambda r: (0, 0)),
            pl.BlockSpec((1, 16), lambda r: (0, 0)),
            pl.BlockSpec((16, 1), lambda r: (0, 0)),
            pl.BlockSpec((1, 1), lambda r: (0, 0)),
        ],
        out_specs=pl.BlockSpec((RBG, 1), lambda r: (r, 0)),
        out_shape=jax.ShapeDtypeStruct((B, 1), jnp.float32),
    )(g, wih, whh, bih, bhh, watt, batt, wp1, bp1, wp2, bp2)


def kernel(x_seq, edge_index_seq, target_indices, W_proj, b_proj, W_g1, b_g1,
           W_g2, b_g2, W_ih, W_hh, b_ih, b_hh, W_att, b_att, W_p1, b_p1,
           W_p2, b_p2):
    xp = jnp.pad(x_seq, ((0, 0), (0, NP - N), (0, 0)))

    src = edge_index_seq[:, 0, :]
    dst = edge_index_seq[:, 1, :]
    padi = (N + (jnp.arange(EP - E, dtype=jnp.int32) % NPAD))[None, :]
    pads = jnp.broadcast_to(padi, (T, EP - E))
    srcp = jnp.concatenate([src, pads], axis=1)
    dstp = jnp.concatenate([dst, pads], axis=1)
    ei = jnp.stack([srcp.reshape(T, NCHT, CH), dstp.reshape(T, NCHT, CH)],
                   axis=2).reshape(T * NCHT, 2, CH)

    dst_off = (dstp + (jnp.arange(T, dtype=jnp.int32) * NP)[:, None])
    degp = _deg_call()(dst_off.reshape(-1, CH))

    bp = b_proj.reshape(1, P)
    bg1 = b_g1.reshape(1, G)
    bg2 = b_g2.reshape(1, G)

    h1s, h2s = [], []
    for t in range(T):
        y1 = _proj_call(t, xp, degp, W_proj, bp, W_g1)
        p = _conv_call(t)(y1, ei)
        h1, y2 = _mid_call(t, p, y1, degp, bg1, W_g2)
        q = _conv_call(t)(y2, ei)
        h2 = _post_call(t, q, y2, degp, bg2)
        h1s.append(h1)
        h2s.append(h2)

    g = _tgt_gather_call()(h1s[0], h1s[1], h1s[2], h1s[3],
                           h2s[0], h2s[1], h2s[2], h2s[3], target_indices)

    out = _gru_call(
        g, W_ih.T, W_hh.T,
        b_ih.reshape(1, 3 * R), b_hh.reshape(1, 3 * R),
        W_att, b_att.reshape(1, 1),
        W_p1, b_p1.reshape(1, 16), W_p2, b_p2.reshape(1, 1))
    return out
